# baseline (speedup 1.0000x reference)
import functools

import jax
import jax.numpy as jnp
from jax import lax
from jax.experimental import pallas as pl
from jax.experimental.pallas import tpu as pltpu
from jax.experimental.pallas import tpu_sc as plsc

N = 100000
B = 256
D = 64
NT = 144
NC = 2
NS = 16
NW = NC * NS
SEG_PER_W = B // NW
CHUNK = 512
STEP = CHUNK - 8
STARTS_PAD = 272


def _pool_body(hid0_hbm, hid1_hbm, starts_hbm, out0_hbm, out1_hbm,
               starts_v, buf, outv):
    cid = lax.axis_index("c")
    sid = lax.axis_index("s")
    w = sid * NC + cid
    pltpu.sync_copy(starts_hbm, starts_v)
    neg_inf = jnp.full((16,), -jnp.inf, dtype=jnp.float32)
    zero = jnp.zeros((16,), dtype=jnp.float32)

    sv = starts_v[pl.ds(w * SEG_PER_W, 16)]
    for src, out_hbm in ((hid0_hbm, out0_hbm), (hid1_hbm, out1_hbm)):
        for si in range(SEG_PER_W):
            lo = sv[si]
            hi = sv[si + 1]
            n = hi - lo
            nch = lax.div(n + (STEP - 1), STEP)

            def row_body(r, accs):
                s0, s1, s2, s3, m0, m1, m2, m3 = accs
                v0 = buf[r, pl.ds(0, 16)]
                v1 = buf[r, pl.ds(16, 16)]
                v2 = buf[r, pl.ds(32, 16)]
                v3 = buf[r, pl.ds(48, 16)]
                return (s0 + v0, s1 + v1, s2 + v2, s3 + v3,
                        jnp.maximum(m0, v0), jnp.maximum(m1, v1),
                        jnp.maximum(m2, v2), jnp.maximum(m3, v3))

            def chunk_body(k, accs):
                c0 = lo + k * STEP
                base = (jnp.minimum(c0, N - CHUNK) // 8) * 8
                a = c0 - base
                b = jnp.minimum(c0 + STEP, hi) - base
                pltpu.sync_copy(src.at[pl.ds(base, CHUNK), :], buf)
                return lax.fori_loop(a, b, row_body, accs)

            init = (zero, zero, zero, zero,
                    neg_inf, neg_inf, neg_inf, neg_inf)
            accs = lax.fori_loop(0, nch, chunk_body, init)
            for j in range(4):
                outv[si, pl.ds(j * 16, 16)] = accs[j]
                outv[si, pl.ds(64 + j * 16, 16)] = accs[4 + j]
        pltpu.sync_copy(outv, out_hbm.at[pl.ds(w * SEG_PER_W, SEG_PER_W), :])


def _segment_pool(hid0, hid1, starts_padded):
    mesh = plsc.VectorSubcoreMesh(core_axis_name="c", subcore_axis_name="s",
                                  num_cores=NC, num_subcores=NS)
    f = pl.kernel(
        _pool_body,
        out_type=(jax.ShapeDtypeStruct((B, 128), jnp.float32),
                  jax.ShapeDtypeStruct((B, 128), jnp.float32)),
        mesh=mesh,
        scratch_types=[
            pltpu.VMEM((STARTS_PAD,), jnp.int32),
            pltpu.VMEM((CHUNK, D), jnp.float32),
            pltpu.VMEM((SEG_PER_W, 128), jnp.float32),
        ],
    )
    return f(hid0, hid1, starts_padded)


def _lrelu(x):
    return jnp.where(x >= 0, x, 0.01 * x)


def _dense_body(p0_ref, p1_ref, slo_ref, shi_ref,
                lc00_ref, lc01_ref, lc10_ref, lc11_ref, bo_ref,
                W1_ref, b1_ref, W2_ref, b2_ref,
                Wa_mean_ref, Wa_max_ref, Wa_l0_ref, Wa_l1_ref, ba_ref,
                Wb_ref, bb_ref, Wo1_ref, bo1_ref, Wo2_ref, bo2_ref,
                out_ref):
    f32 = jnp.float32
    cnt = jnp.maximum((shi_ref[...] - slo_ref[...]).astype(f32), 1.0)

    def mae(x):
        h = _lrelu(jnp.dot(x, W1_ref[...], preferred_element_type=f32)
                   + b1_ref[...])
        return jnp.dot(h, W2_ref[...], preferred_element_type=f32) + b2_ref[...]

    def embed(p_ref, lat0, lat1):
        mean = p_ref[:, 0:64] / cnt
        mx = p_ref[:, 64:128]
        pre = (jnp.dot(mean, Wa_mean_ref[...], preferred_element_type=f32)
               + jnp.dot(mx, Wa_max_ref[...], preferred_element_type=f32)
               + jnp.dot(lat0, Wa_l0_ref[...], preferred_element_type=f32)
               + jnp.dot(lat1, Wa_l1_ref[...], preferred_element_type=f32)
               + ba_ref[...])
        return (jnp.dot(_lrelu(pre), Wb_ref[...], preferred_element_type=f32)
                + bb_ref[...])

    h1 = embed(p0_ref, mae(lc00_ref[...]), mae(lc01_ref[...]))
    h2 = embed(p1_ref, mae(lc10_ref[...]), mae(lc11_ref[...]))
    ho = (jnp.dot(_lrelu(bo_ref[...] * Wo1_ref[...] + bo1_ref[...]),
                  Wo2_ref[...], preferred_element_type=f32) + bo2_ref[...])

    hs = h1 + h2
    qq = (jnp.sum(h1 * h1, axis=1, keepdims=True)
          + jnp.sum(h2 * h2, axis=1, keepdims=True))
    kk = 2.0 * jnp.sum(ho * ho, axis=1, keepdims=True)
    dims = (((1,), (1,)), ((), ()))
    QK = lax.dot_general(hs, ho, dims, preferred_element_type=f32)
    QK2 = lax.dot_general(ho, hs, dims, preferred_element_type=f32)

    rows = lax.broadcasted_iota(jnp.int32, (B, B), 0)
    cols = lax.broadcasted_iota(jnp.int32, (B, B), 1)
    eye = (rows == cols).astype(f32)

    diag_qk = jnp.sum(QK * eye, axis=1, keepdims=True)
    dpos = jnp.sqrt(jnp.maximum(qq + kk - 2.0 * diag_qk, 0.0))

    def ctl(qq_r, kk_c, QKm):
        dneg2 = jnp.maximum(qq_r + kk_c - 2.0 * QKm, 1e-12)
        dneg = jnp.sqrt(dneg2)
        m = (dneg > dpos).astype(f32)
        lam = jnp.exp(0.9 * (jnp.log(dpos) - jnp.log(dneg)))
        return m * (qq_r + lam * (QKm - qq_r)) + (1.0 - m) * QKm

    l_st = ctl(qq, jnp.transpose(kk), QK)
    l_ts = ctl(kk, jnp.transpose(qq), QK2) + eye * (-1e9)

    rmax = jnp.maximum(jnp.max(l_st, axis=1, keepdims=True),
                       jnp.max(l_ts, axis=1, keepdims=True))
    ssum = (jnp.sum(jnp.exp(l_st - rmax), axis=1, keepdims=True)
            + jnp.sum(jnp.exp(l_ts - rmax), axis=1, keepdims=True))
    lse = rmax + jnp.log(ssum)
    picked = jnp.sum(l_st * eye, axis=1, keepdims=True)
    out_ref[...] = jnp.sum(lse - picked, axis=0, keepdims=True) * (1.0 / B)


def kernel(hidden_atom, hidden_atom_1, lattice_coord, lattice_coord1,
           batch_output, batch_ids, W1, b1, W2, b2, Wa, ba, Wb, bb,
           Wo1, bo1, Wo2, bo2):
    ids = batch_ids.astype(jnp.int32)
    starts = jnp.searchsorted(
        ids, jnp.arange(B + 1, dtype=jnp.int32)).astype(jnp.int32)
    starts_padded = jnp.concatenate(
        [starts, jnp.full((STARTS_PAD - (B + 1),), N, dtype=jnp.int32)])

    pooled0, pooled1 = _segment_pool(hidden_atom, hidden_atom_1, starts_padded)

    dense = pl.pallas_call(
        _dense_body,
        out_shape=jax.ShapeDtypeStruct((1, 1), jnp.float32),
    )
    loss = dense(
        pooled0, pooled1,
        starts[:B, None], starts[1:B + 1, None],
        lattice_coord[:, 0, :], lattice_coord[:, 1, :],
        lattice_coord1[:, 0, :], lattice_coord1[:, 1, :],
        batch_output[:, None],
        W1, b1[None, :], W2, b2[None, :],
        Wa[0:64], Wa[64:128], Wa[128:136], Wa[136:144], ba[None, :],
        Wb, bb[None, :], Wo1, bo1[None, :], Wo2, bo2[None, :],
    )
    return jnp.reshape(loss, ())


# speedup vs baseline: 1.1175x; 1.1175x over previous
import functools

import jax
import jax.numpy as jnp
from jax import lax
from jax.experimental import pallas as pl
from jax.experimental.pallas import tpu as pltpu
from jax.experimental.pallas import tpu_sc as plsc

N = 100000
B = 256
D = 64
NT = 144
NC = 2
NS = 16
NW = NC * NS
SEG_PER_W = B // NW
CHUNK = 256
STEP = CHUNK - 8
STARTS_PAD = 272


def _pool_body(hid0_hbm, hid1_hbm, starts_hbm, out0_hbm, out1_hbm,
               starts_v, buf0, buf1, outv, sem0, sem1):
    cid = lax.axis_index("c")
    sid = lax.axis_index("s")
    w = sid * NC + cid
    pltpu.sync_copy(starts_hbm, starts_v)
    neg_inf = jnp.full((16,), -jnp.inf, dtype=jnp.float32)
    zero = jnp.zeros((16,), dtype=jnp.float32)
    init = (zero, zero, zero, zero, neg_inf, neg_inf, neg_inf, neg_inf)

    sv = starts_v[pl.ds(w * SEG_PER_W, 16)]
    r0 = sv[0]
    r8 = sv[SEG_PER_W]
    nch = lax.div(r8 - r0 + (STEP - 1), STEP)
    npair = lax.div(nch + 1, 2)

    def base_of(c0):
        return (jnp.minimum(c0, N - CHUNK) // 8) * 8

    for src, out_hbm in ((hid0_hbm, out0_hbm), (hid1_hbm, out1_hbm)):
        for si in range(SEG_PER_W):
            for j in range(4):
                outv[si, pl.ds(j * 16, 16)] = zero
                outv[si, pl.ds(64 + j * 16, 16)] = neg_inf

        def process(bufref, c0, accs):
            base = base_of(c0)
            for si in range(SEG_PER_W):
                lo = sv[si]
                hi = sv[si + 1]
                a = jnp.maximum(lo, c0) - base
                b = jnp.maximum(jnp.minimum(hi, c0 + STEP) - base, a)

                def row_body(r, accs2):
                    s0, s1, s2, s3, m0, m1, m2, m3 = accs2
                    v0 = bufref[r, pl.ds(0, 16)]
                    v1 = bufref[r, pl.ds(16, 16)]
                    v2 = bufref[r, pl.ds(32, 16)]
                    v3 = bufref[r, pl.ds(48, 16)]
                    return (s0 + v0, s1 + v1, s2 + v2, s3 + v3,
                            jnp.maximum(m0, v0), jnp.maximum(m1, v1),
                            jnp.maximum(m2, v2), jnp.maximum(m3, v3))

                accs = lax.fori_loop(a, b, row_body, accs)
                closes = jnp.logical_and(hi > c0, hi <= c0 + STEP)

                @pl.when(closes)
                def _():
                    for j in range(4):
                        outv[si, pl.ds(j * 16, 16)] = accs[j]
                        outv[si, pl.ds(64 + j * 16, 16)] = accs[4 + j]

                accs = tuple(jnp.where(closes, i_v, a_v)
                             for i_v, a_v in zip(init, accs))
            return accs

        def copy(c0, bufref, sem):
            return pltpu.make_async_copy(
                src.at[pl.ds(base_of(c0), CHUNK), :], bufref, sem)

        copy(r0, buf0, sem0).start()

        def pair_body(kk, accs):
            c0a = r0 + (2 * kk) * STEP
            c0b = c0a + STEP
            c0c = c0b + STEP
            copy(c0b, buf1, sem1).start()
            copy(c0a, buf0, sem0).wait()
            accs = process(buf0, c0a, accs)
            copy(c0c, buf0, sem0).start()
            copy(c0b, buf1, sem1).wait()
            return process(buf1, c0b, accs)

        lax.fori_loop(0, npair, pair_body, init)
        copy(r0 + 2 * npair * STEP, buf0, sem0).wait()
        pltpu.sync_copy(outv, out_hbm.at[pl.ds(w * SEG_PER_W, SEG_PER_W), :])


def _segment_pool(hid0, hid1, starts_padded):
    mesh = plsc.VectorSubcoreMesh(core_axis_name="c", subcore_axis_name="s",
                                  num_cores=NC, num_subcores=NS)
    f = pl.kernel(
        _pool_body,
        out_type=(jax.ShapeDtypeStruct((B, 128), jnp.float32),
                  jax.ShapeDtypeStruct((B, 128), jnp.float32)),
        mesh=mesh,
        scratch_types=[
            pltpu.VMEM((STARTS_PAD,), jnp.int32),
            pltpu.VMEM((CHUNK, D), jnp.float32),
            pltpu.VMEM((CHUNK, D), jnp.float32),
            pltpu.VMEM((SEG_PER_W, 128), jnp.float32),
            pltpu.SemaphoreType.DMA,
            pltpu.SemaphoreType.DMA,
        ],
    )
    return f(hid0, hid1, starts_padded)


def _lrelu(x):
    return jnp.where(x >= 0, x, 0.01 * x)


def _dense_body(p0_ref, p1_ref, slo_ref, shi_ref,
                lc00_ref, lc01_ref, lc10_ref, lc11_ref, bo_ref,
                W1_ref, b1_ref, W2_ref, b2_ref,
                Wa_mean_ref, Wa_max_ref, Wa_l0_ref, Wa_l1_ref, ba_ref,
                Wb_ref, bb_ref, Wo1_ref, bo1_ref, Wo2_ref, bo2_ref,
                out_ref):
    f32 = jnp.float32
    cnt = jnp.maximum((shi_ref[...] - slo_ref[...]).astype(f32), 1.0)

    def mae(x):
        h = _lrelu(jnp.dot(x, W1_ref[...], preferred_element_type=f32)
                   + b1_ref[...])
        return jnp.dot(h, W2_ref[...], preferred_element_type=f32) + b2_ref[...]

    def embed(p_ref, lat0, lat1):
        mean = p_ref[:, 0:64] / cnt
        mx = p_ref[:, 64:128]
        pre = (jnp.dot(mean, Wa_mean_ref[...], preferred_element_type=f32)
               + jnp.dot(mx, Wa_max_ref[...], preferred_element_type=f32)
               + jnp.dot(lat0, Wa_l0_ref[...], preferred_element_type=f32)
               + jnp.dot(lat1, Wa_l1_ref[...], preferred_element_type=f32)
               + ba_ref[...])
        return (jnp.dot(_lrelu(pre), Wb_ref[...], preferred_element_type=f32)
                + bb_ref[...])

    h1 = embed(p0_ref, mae(lc00_ref[...]), mae(lc01_ref[...]))
    h2 = embed(p1_ref, mae(lc10_ref[...]), mae(lc11_ref[...]))
    ho = (jnp.dot(_lrelu(bo_ref[...] * Wo1_ref[...] + bo1_ref[...]),
                  Wo2_ref[...], preferred_element_type=f32) + bo2_ref[...])

    hs = h1 + h2
    qq = (jnp.sum(h1 * h1, axis=1, keepdims=True)
          + jnp.sum(h2 * h2, axis=1, keepdims=True))
    kk = 2.0 * jnp.sum(ho * ho, axis=1, keepdims=True)
    dims = (((1,), (1,)), ((), ()))
    QK = lax.dot_general(hs, ho, dims, preferred_element_type=f32)
    QK2 = lax.dot_general(ho, hs, dims, preferred_element_type=f32)

    rows = lax.broadcasted_iota(jnp.int32, (B, B), 0)
    cols = lax.broadcasted_iota(jnp.int32, (B, B), 1)
    eye = (rows == cols).astype(f32)

    diag_qk = jnp.sum(QK * eye, axis=1, keepdims=True)
    dpos = jnp.sqrt(jnp.maximum(qq + kk - 2.0 * diag_qk, 0.0))

    def ctl(qq_r, kk_c, QKm):
        dneg2 = jnp.maximum(qq_r + kk_c - 2.0 * QKm, 1e-12)
        dneg = jnp.sqrt(dneg2)
        m = (dneg > dpos).astype(f32)
        lam = jnp.exp(0.9 * (jnp.log(dpos) - jnp.log(dneg)))
        return m * (qq_r + lam * (QKm - qq_r)) + (1.0 - m) * QKm

    l_st = ctl(qq, jnp.transpose(kk), QK)
    l_ts = ctl(kk, jnp.transpose(qq), QK2) + eye * (-1e9)

    rmax = jnp.maximum(jnp.max(l_st, axis=1, keepdims=True),
                       jnp.max(l_ts, axis=1, keepdims=True))
    ssum = (jnp.sum(jnp.exp(l_st - rmax), axis=1, keepdims=True)
            + jnp.sum(jnp.exp(l_ts - rmax), axis=1, keepdims=True))
    lse = rmax + jnp.log(ssum)
    picked = jnp.sum(l_st * eye, axis=1, keepdims=True)
    out_ref[...] = jnp.sum(lse - picked, axis=0, keepdims=True) * (1.0 / B)


def kernel(hidden_atom, hidden_atom_1, lattice_coord, lattice_coord1,
           batch_output, batch_ids, W1, b1, W2, b2, Wa, ba, Wb, bb,
           Wo1, bo1, Wo2, bo2):
    ids = batch_ids.astype(jnp.int32)
    starts = jnp.searchsorted(
        ids, jnp.arange(B + 1, dtype=jnp.int32)).astype(jnp.int32)
    starts_padded = jnp.concatenate(
        [starts, jnp.full((STARTS_PAD - (B + 1),), N, dtype=jnp.int32)])

    pooled0, pooled1 = _segment_pool(hidden_atom, hidden_atom_1, starts_padded)

    dense = pl.pallas_call(
        _dense_body,
        out_shape=jax.ShapeDtypeStruct((1, 1), jnp.float32),
    )
    loss = dense(
        pooled0, pooled1,
        starts[:B, None], starts[1:B + 1, None],
        lattice_coord[:, 0, :], lattice_coord[:, 1, :],
        lattice_coord1[:, 0, :], lattice_coord1[:, 1, :],
        batch_output[:, None],
        W1, b1[None, :], W2, b2[None, :],
        Wa[0:64], Wa[64:128], Wa[128:136], Wa[136:144], ba[None, :],
        Wb, bb[None, :], Wo1, bo1[None, :], Wo2, bo2[None, :],
    )
    return jnp.reshape(loss, ())


# speedup vs baseline: 1.4840x; 1.3280x over previous
import functools

import jax
import jax.numpy as jnp
from jax import lax
from jax.experimental import pallas as pl
from jax.experimental.pallas import tpu as pltpu
from jax.experimental.pallas import tpu_sc as plsc

N = 100000
B = 256
D = 64
NT = 144
NC = 2
NS = 16
NW = NC * NS
SEG_PER_W = B // NW
CHUNK = 256
STEP = CHUNK - 8
STARTS_PAD = 272


def _pool_body(hid0_hbm, hid1_hbm, ids_hbm, out0_hbm, out1_hbm, starts_hbm,
               gat_v, svv, buf0, buf1, outv, sem0, sem1):
    cid = lax.axis_index("c")
    sid = lax.axis_index("s")
    w = sid * NC + cid
    neg_inf = jnp.full((16,), -jnp.inf, dtype=jnp.float32)
    zero = jnp.zeros((16,), dtype=jnp.float32)
    init = (zero, zero, zero, zero, neg_inf, neg_inf, neg_inf, neg_inf)

    tgt = w * SEG_PER_W + lax.iota(jnp.int32, 16)
    blo = jnp.zeros((16,), jnp.int32)
    bhi = jnp.full((16,), N, jnp.int32)
    for _ in range(17):
        mid = lax.shift_right_logical(blo + bhi, 1)
        pltpu.async_copy(ids_hbm.at[mid], gat_v, sem0).wait()
        pred = gat_v[...] >= tgt
        bhi = jnp.where(pred, mid, bhi)
        blo = jnp.where(pred, blo, mid + 1)
    sv = blo
    svv[0, pl.ds(0, 16)] = sv
    pltpu.sync_copy(svv, starts_hbm.at[pl.ds(w, 1), :])
    r0 = sv[0]
    r8 = sv[SEG_PER_W]
    nch = lax.div(r8 - r0 + (STEP - 1), STEP)
    npair = lax.div(nch + 1, 2)

    def base_of(c0):
        return (jnp.minimum(c0, N - CHUNK) // 8) * 8

    for src, out_hbm in ((hid0_hbm, out0_hbm), (hid1_hbm, out1_hbm)):
        for si in range(SEG_PER_W):
            for j in range(4):
                outv[si, pl.ds(j * 16, 16)] = zero
                outv[si, pl.ds(64 + j * 16, 16)] = neg_inf

        def process(bufref, c0, accs):
            base = base_of(c0)
            for si in range(SEG_PER_W):
                lo = sv[si]
                hi = sv[si + 1]
                a = jnp.maximum(lo, c0) - base
                b = jnp.maximum(jnp.minimum(hi, c0 + STEP) - base, a)

                def row_body(r, accs2):
                    s0, s1, s2, s3, m0, m1, m2, m3 = accs2
                    v0 = bufref[r, pl.ds(0, 16)]
                    v1 = bufref[r, pl.ds(16, 16)]
                    v2 = bufref[r, pl.ds(32, 16)]
                    v3 = bufref[r, pl.ds(48, 16)]
                    return (s0 + v0, s1 + v1, s2 + v2, s3 + v3,
                            jnp.maximum(m0, v0), jnp.maximum(m1, v1),
                            jnp.maximum(m2, v2), jnp.maximum(m3, v3))

                accs = lax.fori_loop(a, b, row_body, accs)
                closes = jnp.logical_and(hi > c0, hi <= c0 + STEP)

                @pl.when(closes)
                def _():
                    for j in range(4):
                        outv[si, pl.ds(j * 16, 16)] = accs[j]
                        outv[si, pl.ds(64 + j * 16, 16)] = accs[4 + j]

                accs = tuple(jnp.where(closes, i_v, a_v)
                             for i_v, a_v in zip(init, accs))
            return accs

        def copy(c0, bufref, sem):
            return pltpu.make_async_copy(
                src.at[pl.ds(base_of(c0), CHUNK), :], bufref, sem)

        copy(r0, buf0, sem0).start()

        def pair_body(kk, accs):
            c0a = r0 + (2 * kk) * STEP
            c0b = c0a + STEP
            c0c = c0b + STEP
            copy(c0b, buf1, sem1).start()
            copy(c0a, buf0, sem0).wait()
            accs = process(buf0, c0a, accs)
            copy(c0c, buf0, sem0).start()
            copy(c0b, buf1, sem1).wait()
            return process(buf1, c0b, accs)

        lax.fori_loop(0, npair, pair_body, init)
        copy(r0 + 2 * npair * STEP, buf0, sem0).wait()
        pltpu.sync_copy(outv, out_hbm.at[pl.ds(w * SEG_PER_W, SEG_PER_W), :])


def _segment_pool(hid0, hid1, ids):
    mesh = plsc.VectorSubcoreMesh(core_axis_name="c", subcore_axis_name="s",
                                  num_cores=NC, num_subcores=NS)
    f = pl.kernel(
        _pool_body,
        out_type=(jax.ShapeDtypeStruct((B, 128), jnp.float32),
                  jax.ShapeDtypeStruct((B, 128), jnp.float32),
                  jax.ShapeDtypeStruct((NW, 16), jnp.int32)),
        mesh=mesh,
        scratch_types=[
            pltpu.VMEM((16,), jnp.int32),
            pltpu.VMEM((1, 16), jnp.int32),
            pltpu.VMEM((CHUNK, D), jnp.float32),
            pltpu.VMEM((CHUNK, D), jnp.float32),
            pltpu.VMEM((SEG_PER_W, 128), jnp.float32),
            pltpu.SemaphoreType.DMA,
            pltpu.SemaphoreType.DMA,
        ],
    )
    return f(hid0, hid1, ids)


def _lrelu(x):
    return jnp.where(x >= 0, x, 0.01 * x)


def _dense_body(p0_ref, p1_ref, slo_ref, shi_ref,
                lc00_ref, lc01_ref, lc10_ref, lc11_ref, bo_ref,
                W1_ref, b1_ref, W2_ref, b2_ref,
                Wa_mean_ref, Wa_max_ref, Wa_l0_ref, Wa_l1_ref, ba_ref,
                Wb_ref, bb_ref, Wo1_ref, bo1_ref, Wo2_ref, bo2_ref,
                out_ref):
    f32 = jnp.float32
    cnt = jnp.maximum((shi_ref[...] - slo_ref[...]).astype(f32), 1.0)

    def mae(x):
        h = _lrelu(jnp.dot(x, W1_ref[...], preferred_element_type=f32)
                   + b1_ref[...])
        return jnp.dot(h, W2_ref[...], preferred_element_type=f32) + b2_ref[...]

    def embed(p_ref, lat0, lat1):
        mean = p_ref[:, 0:64] / cnt
        mx = p_ref[:, 64:128]
        pre = (jnp.dot(mean, Wa_mean_ref[...], preferred_element_type=f32)
               + jnp.dot(mx, Wa_max_ref[...], preferred_element_type=f32)
               + jnp.dot(lat0, Wa_l0_ref[...], preferred_element_type=f32)
               + jnp.dot(lat1, Wa_l1_ref[...], preferred_element_type=f32)
               + ba_ref[...])
        return (jnp.dot(_lrelu(pre), Wb_ref[...], preferred_element_type=f32)
                + bb_ref[...])

    h1 = embed(p0_ref, mae(lc00_ref[...]), mae(lc01_ref[...]))
    h2 = embed(p1_ref, mae(lc10_ref[...]), mae(lc11_ref[...]))
    ho = (jnp.dot(_lrelu(bo_ref[...] * Wo1_ref[...] + bo1_ref[...]),
                  Wo2_ref[...], preferred_element_type=f32) + bo2_ref[...])

    hs = h1 + h2
    qq = (jnp.sum(h1 * h1, axis=1, keepdims=True)
          + jnp.sum(h2 * h2, axis=1, keepdims=True))
    kk = 2.0 * jnp.sum(ho * ho, axis=1, keepdims=True)
    dims = (((1,), (1,)), ((), ()))
    QK = lax.dot_general(hs, ho, dims, preferred_element_type=f32)
    QK2 = lax.dot_general(ho, hs, dims, preferred_element_type=f32)

    rows = lax.broadcasted_iota(jnp.int32, (B, B), 0)
    cols = lax.broadcasted_iota(jnp.int32, (B, B), 1)
    eye = (rows == cols).astype(f32)

    diag_qk = jnp.sum(QK * eye, axis=1, keepdims=True)
    dpos = jnp.sqrt(jnp.maximum(qq + kk - 2.0 * diag_qk, 0.0))

    def ctl(qq_r, kk_c, QKm):
        dneg2 = jnp.maximum(qq_r + kk_c - 2.0 * QKm, 1e-12)
        dneg = jnp.sqrt(dneg2)
        m = (dneg > dpos).astype(f32)
        lam = jnp.exp(0.9 * (jnp.log(dpos) - jnp.log(dneg)))
        return m * (qq_r + lam * (QKm - qq_r)) + (1.0 - m) * QKm

    l_st = ctl(qq, jnp.transpose(kk), QK)
    l_ts = ctl(kk, jnp.transpose(qq), QK2) + eye * (-1e9)

    rmax = jnp.maximum(jnp.max(l_st, axis=1, keepdims=True),
                       jnp.max(l_ts, axis=1, keepdims=True))
    ssum = (jnp.sum(jnp.exp(l_st - rmax), axis=1, keepdims=True)
            + jnp.sum(jnp.exp(l_ts - rmax), axis=1, keepdims=True))
    lse = rmax + jnp.log(ssum)
    picked = jnp.sum(l_st * eye, axis=1, keepdims=True)
    out_ref[...] = jnp.sum(lse - picked, axis=0, keepdims=True) * (1.0 / B)


def kernel(hidden_atom, hidden_atom_1, lattice_coord, lattice_coord1,
           batch_output, batch_ids, W1, b1, W2, b2, Wa, ba, Wb, bb,
           Wo1, bo1, Wo2, bo2):
    ids = batch_ids.astype(jnp.int32)
    pooled0, pooled1, starts_w = _segment_pool(hidden_atom, hidden_atom_1, ids)
    slo = starts_w[:, :SEG_PER_W].reshape(B, 1)
    shi = starts_w[:, 1:SEG_PER_W + 1].reshape(B, 1)

    dense = pl.pallas_call(
        _dense_body,
        out_shape=jax.ShapeDtypeStruct((1, 1), jnp.float32),
    )
    loss = dense(
        pooled0, pooled1,
        slo, shi,
        lattice_coord[:, 0, :], lattice_coord[:, 1, :],
        lattice_coord1[:, 0, :], lattice_coord1[:, 1, :],
        batch_output[:, None],
        W1, b1[None, :], W2, b2[None, :],
        Wa[0:64], Wa[64:128], Wa[128:136], Wa[136:144], ba[None, :],
        Wb, bb[None, :], Wo1, bo1[None, :], Wo2, bo2[None, :],
    )
    return jnp.reshape(loss, ())
